# baseline (speedup 1.0000x reference)
import functools

import jax
import jax.numpy as jnp
from jax import lax
from jax.experimental import pallas as pl
from jax.experimental.pallas import tpu as pltpu
from jax.experimental.pallas import tpu_sc as plsc

N = 10000
N_PAD = 10240
E = 320000
DE = 16
NW = 32
E_T = E // NW
NB = 5
E_B = E_T // NB
NSUB = 25
CH = E_B // NSUB
ROWS_T = N_PAD // 16


def _sc_scatter(attr4, col4):
    mesh = plsc.VectorSubcoreMesh(core_axis_name="c", subcore_axis_name="s")

    @functools.partial(
        pl.kernel,
        mesh=mesh,
        out_type=(
            jax.ShapeDtypeStruct((2, N_PAD, DE), jnp.float32),
            jax.ShapeDtypeStruct((2, N_PAD), jnp.float32),
        ),
        scratch_types=[
            pltpu.VMEM((E_B, DE), jnp.float32),
            pltpu.VMEM((NSUB, CH), jnp.int32),
            pltpu.VMEM((N_PAD,), jnp.float32),
            pltpu.VMEM_SHARED((N_PAD, DE), jnp.float32),
            pltpu.VMEM_SHARED((16, N_PAD), jnp.float32),
            pltpu.VMEM((16, ROWS_T), jnp.float32),
            pltpu.VMEM((ROWS_T,), jnp.float32),
        ],
        compiler_params=pltpu.CompilerParams(
            needs_layout_passes=False, use_tc_tiling_on_sc=False
        ),
    )
    def k(
        attr_hbm, col_hbm, sums_hbm, cnt_hbm,
        attr_buf, idx_buf, cnt_acc, acc, shared_cnt, cbuf, obuf,
    ):
        c = lax.axis_index("c")
        s = lax.axis_index("s")
        wid = c * 16 + s
        zero16 = jnp.zeros((16,), jnp.float32)

        def zero_counts(i, carry):
            cnt_acc[pl.ds(i * 16, 16)] = zero16
            return carry

        lax.fori_loop(0, N_PAD // 16, zero_counts, 0)

        def zero_rows(i, carry):
            attr_buf[i, :] = zero16
            return carry

        lax.fori_loop(0, ROWS_T, zero_rows, 0)
        pltpu.sync_copy(
            attr_buf.at[pl.ds(0, ROWS_T)], acc.at[pl.ds(s * ROWS_T, ROWS_T)]
        )
        plsc.subcore_barrier()

        ones16 = jnp.ones((16,), jnp.float32)
        for b in range(NB):
            pltpu.sync_copy(
                attr_hbm.at[pl.ds(wid * E_T + b * E_B, E_B)], attr_buf
            )
            pltpu.sync_copy(col_hbm.at[wid, b], idx_buf)

            def scatter_chunk(j, carry):
                pltpu.sync_copy(
                    attr_buf.at[pl.ds(j * CH, CH)],
                    acc.at[idx_buf.at[j]],
                    add=True,
                )
                return carry

            lax.fori_loop(0, NSUB, scatter_chunk, 0)

            def count_chunk(j, carry):
                for kk in range(CH // 16):
                    v = idx_buf[j, pl.ds(kk * 16, 16)]
                    plsc.addupdate_scatter(cnt_acc, [v], ones16)
                return carry

            lax.fori_loop(0, NSUB, count_chunk, 0)

        pltpu.sync_copy(cnt_acc, shared_cnt.at[s])
        plsc.subcore_barrier()
        pltpu.sync_copy(
            acc.at[pl.ds(s * ROWS_T, ROWS_T)],
            sums_hbm.at[c, pl.ds(s * ROWS_T, ROWS_T)],
        )
        for r in range(16):
            pltpu.sync_copy(
                shared_cnt.at[r, pl.ds(s * ROWS_T, ROWS_T)], cbuf.at[r]
            )

        def cnt_reduce(v, carry):
            tot = cbuf[0, pl.ds(v * 16, 16)]
            for r in range(1, 16):
                tot = tot + cbuf[r, pl.ds(v * 16, 16)]
            obuf[pl.ds(v * 16, 16)] = tot
            return carry

        lax.fori_loop(0, ROWS_T // 16, cnt_reduce, 0)
        pltpu.sync_copy(obuf, cnt_hbm.at[c, pl.ds(s * ROWS_T, ROWS_T)])

    return k(attr4, col4)


def _mlp(x, sums2, cnts, W1a, W1b, b1, W2, b2):
    BR = 1000

    def body(x_ref, s_ref, c_ref, w1a, w1b, b1r, w2, b2r, o_ref):
        cnt = c_ref[0, :, 0] + c_ref[1, :, 0]
        sm = s_ref[0] + s_ref[1]
        mean = sm / jnp.maximum(cnt, 1.0)[:, None]
        h = jnp.maximum(
            jnp.dot(x_ref[...], w1a[...], preferred_element_type=jnp.float32)
            + jnp.dot(mean, w1b[...], preferred_element_type=jnp.float32)
            + b1r[...],
            0.0,
        )
        o_ref[...] = jnp.dot(h, w2[...], preferred_element_type=jnp.float32) + b2r[...]

    return pl.pallas_call(
        body,
        grid=(N // BR,),
        in_specs=[
            pl.BlockSpec((BR, 128), lambda i: (i, 0)),
            pl.BlockSpec((2, BR, DE), lambda i: (0, i, 0)),
            pl.BlockSpec((2, BR, 1), lambda i: (0, i, 0)),
            pl.BlockSpec((128, 128), lambda i: (0, 0)),
            pl.BlockSpec((DE, 128), lambda i: (0, 0)),
            pl.BlockSpec((1, 128), lambda i: (0, 0)),
            pl.BlockSpec((128, 128), lambda i: (0, 0)),
            pl.BlockSpec((1, 128), lambda i: (0, 0)),
        ],
        out_specs=pl.BlockSpec((BR, 128), lambda i: (i, 0)),
        out_shape=jax.ShapeDtypeStruct((N, 128), jnp.float32),
    )(x, sums2, cnts, W1a, W1b, b1.reshape(1, 128), W2, b2.reshape(1, 128))


def kernel(x, edge_index, edge_attr, W1, b1, W2, b2):
    col = edge_index[1]
    col4 = col.reshape(NW, NB, NSUB, CH)
    sums2, cnts = _sc_scatter(edge_attr, col4)
    cnts3 = cnts.reshape(2, N_PAD, 1)
    return _mlp(x, sums2, cnts3, W1[:128], W1[128:], b1, W2, b2)


# speedup vs baseline: 1.1456x; 1.1456x over previous
import functools

import jax
import jax.numpy as jnp
from jax import lax
from jax.experimental import pallas as pl
from jax.experimental.pallas import tpu as pltpu
from jax.experimental.pallas import tpu_sc as plsc

N = 10000
N_PAD = 10240
E = 320000
DE = 16
NW = 32
E_T = E // NW
NB = 5
E_B = E_T // NB
NSUB = 25
CH = E_B // NSUB
ROWS_T = N_PAD // 16


def _sc_scatter(attr2, col4):
    mesh = plsc.VectorSubcoreMesh(core_axis_name="c", subcore_axis_name="s")

    @functools.partial(
        pl.kernel,
        mesh=mesh,
        out_type=(
            jax.ShapeDtypeStruct((2, N_PAD, DE), jnp.float32),
            jax.ShapeDtypeStruct((NW * N_PAD,), jnp.float32),
        ),
        scratch_types=[
            pltpu.VMEM((2, E_B, DE), jnp.float32),
            pltpu.VMEM((2, NSUB, CH), jnp.int32),
            pltpu.VMEM((N_PAD,), jnp.float32),
            pltpu.VMEM_SHARED((N_PAD, DE), jnp.float32),
            pltpu.SemaphoreType.DMA,
            pltpu.SemaphoreType.DMA,
            pltpu.SemaphoreType.DMA,
            pltpu.SemaphoreType.DMA,
            pltpu.SemaphoreType.DMA,
        ],
        compiler_params=pltpu.CompilerParams(
            needs_layout_passes=False, use_tc_tiling_on_sc=False
        ),
    )
    def k(
        attr_hbm, col_hbm, sums_hbm, cnt_hbm,
        attr_buf, idx_buf, cnt_acc, acc, sa0, sa1, si0, si1, ssc,
    ):
        c = lax.axis_index("c")
        s = lax.axis_index("s")
        wid = c * 16 + s
        sem_a = (sa0, sa1)
        sem_i = (si0, si1)
        zero16 = jnp.zeros((16,), jnp.float32)

        def zero_counts(i, carry):
            cnt_acc[pl.ds(i * 16, 16)] = zero16
            return carry

        lax.fori_loop(0, N_PAD // 16, zero_counts, 0)

        def zero_rows(i, carry):
            attr_buf[0, i, :] = zero16
            return carry

        lax.fori_loop(0, ROWS_T, zero_rows, 0)
        pltpu.sync_copy(
            attr_buf.at[0, pl.ds(0, ROWS_T)], acc.at[pl.ds(s * ROWS_T, ROWS_T)]
        )
        plsc.subcore_barrier()

        def start_in(b):
            slot = b % 2
            ha = pltpu.async_copy(
                attr_hbm.at[pl.ds(wid * E_T + b * E_B, E_B)],
                attr_buf.at[slot],
                sem_a[slot],
            )
            hi = pltpu.async_copy(
                col_hbm.at[wid, b], idx_buf.at[slot], sem_i[slot]
            )
            return ha, hi

        ones16 = jnp.ones((16,), jnp.float32)
        pending = {0: start_in(0)}
        for b in range(NB):
            slot = b % 2
            ha, hi = pending.pop(b)
            ha.wait()
            hi.wait()
            if b + 1 < NB:
                pending[b + 1] = start_in(b + 1)

            scat = [
                pltpu.async_copy(
                    attr_buf.at[slot, pl.ds(j * CH, CH)],
                    acc.at[idx_buf.at[slot, j]],
                    ssc,
                    add=True,
                )
                for j in range(NSUB)
            ]

            def count_chunk(j, carry):
                for kk in range(CH // 16):
                    v = idx_buf[slot, j, pl.ds(kk * 16, 16)]
                    plsc.addupdate_scatter(cnt_acc, [v], ones16)
                return carry

            lax.fori_loop(0, NSUB, count_chunk, 0)
            for h in scat:
                h.wait()

        plsc.subcore_barrier()
        pltpu.sync_copy(
            acc.at[pl.ds(s * ROWS_T, ROWS_T)],
            sums_hbm.at[c, pl.ds(s * ROWS_T, ROWS_T)],
        )
        pltpu.sync_copy(cnt_acc, cnt_hbm.at[pl.ds(wid * N_PAD, N_PAD)])

    return k(attr2, col4)


def _mlp_pre(x, W1a, b1):
    BR = 1000

    def body(x_ref, w1a, b1r, o_ref):
        o_ref[...] = (
            jnp.dot(x_ref[...], w1a[...], preferred_element_type=jnp.float32)
            + b1r[...]
        )

    return pl.pallas_call(
        body,
        grid=(N // BR,),
        in_specs=[
            pl.BlockSpec((BR, 128), lambda i: (i, 0)),
            pl.BlockSpec((128, 128), lambda i: (0, 0)),
            pl.BlockSpec((1, 128), lambda i: (0, 0)),
        ],
        out_specs=pl.BlockSpec((BR, 128), lambda i: (i, 0)),
        out_shape=jax.ShapeDtypeStruct((N, 128), jnp.float32),
    )(x, W1a, b1.reshape(1, 128))


def _mlp_post(pre, sums2, cnts_t, W1b, W2, b2):
    BR = 1000

    def body(p_ref, s_ref, c_ref, w1b, w2, b2r, o_ref):
        cnt = jnp.sum(c_ref[...], axis=1)
        sm = s_ref[0] + s_ref[1]
        mean = sm / jnp.maximum(cnt, 1.0)[:, None]
        h = jnp.maximum(
            p_ref[...]
            + jnp.dot(mean, w1b[...], preferred_element_type=jnp.float32),
            0.0,
        )
        o_ref[...] = jnp.dot(h, w2[...], preferred_element_type=jnp.float32) + b2r[...]

    return pl.pallas_call(
        body,
        grid=(N // BR,),
        in_specs=[
            pl.BlockSpec((BR, 128), lambda i: (i, 0)),
            pl.BlockSpec((2, BR, DE), lambda i: (0, i, 0)),
            pl.BlockSpec((BR, NW), lambda i: (i, 0)),
            pl.BlockSpec((DE, 128), lambda i: (0, 0)),
            pl.BlockSpec((128, 128), lambda i: (0, 0)),
            pl.BlockSpec((1, 128), lambda i: (0, 0)),
        ],
        out_specs=pl.BlockSpec((BR, 128), lambda i: (i, 0)),
        out_shape=jax.ShapeDtypeStruct((N, 128), jnp.float32),
    )(pre, sums2, cnts_t, W1b, W2, b2.reshape(1, 128))


def kernel(x, edge_index, edge_attr, W1, b1, W2, b2):
    col = edge_index[1]
    col4 = col.reshape(NW, NB, NSUB, CH)
    pre = _mlp_pre(x, W1[:128], b1)
    sums2, cnts = _sc_scatter(edge_attr, col4)
    cnts_t = cnts.reshape(NW, N_PAD).T
    return _mlp_post(pre, sums2, cnts_t, W1[128:], W2, b2)


# speedup vs baseline: 1.1576x; 1.0105x over previous
import functools

import jax
import jax.numpy as jnp
from jax import lax
from jax.experimental import pallas as pl
from jax.experimental.pallas import tpu as pltpu
from jax.experimental.pallas import tpu_sc as plsc

N = 10000
N_PAD = 10240
E = 320000
DE = 16
NW = 32
E_T = E // NW
NB = 5
E_B = E_T // NB
NSUB = 25
CH = E_B // NSUB
ROWS_T = N_PAD // 16


def _sc_scatter(attr2, col4):
    mesh = plsc.VectorSubcoreMesh(core_axis_name="c", subcore_axis_name="s")

    @functools.partial(
        pl.kernel,
        mesh=mesh,
        out_type=(
            jax.ShapeDtypeStruct((2, N_PAD, DE), jnp.float32),
            jax.ShapeDtypeStruct((N // 1000, NW, 1000), jnp.float32),
        ),
        scratch_types=[
            pltpu.VMEM((3, E_B, DE), jnp.float32),
            pltpu.VMEM((3, NSUB, CH), jnp.int32),
            pltpu.VMEM((N_PAD,), jnp.float32),
            pltpu.VMEM_SHARED((N_PAD, DE), jnp.float32),
            pltpu.SemaphoreType.DMA,
            pltpu.SemaphoreType.DMA,
            pltpu.SemaphoreType.DMA,
        ],
        compiler_params=pltpu.CompilerParams(
            needs_layout_passes=False, use_tc_tiling_on_sc=False
        ),
    )
    def k(
        attr_hbm, col_hbm, sums_hbm, cnt_hbm,
        attr_buf, idx_buf, cnt_acc, acc, sem_a, sem_i, ssc,
    ):
        c = lax.axis_index("c")
        s = lax.axis_index("s")
        wid = c * 16 + s
        zero16 = jnp.zeros((16,), jnp.float32)

        def zero_counts(i, carry):
            cnt_acc[pl.ds(i * 16, 16)] = zero16
            return carry

        lax.fori_loop(0, N_PAD // 16, zero_counts, 0)

        def zero_rows(i, carry):
            attr_buf[0, i, :] = zero16
            return carry

        lax.fori_loop(0, ROWS_T, zero_rows, 0)
        pltpu.sync_copy(
            attr_buf.at[0, pl.ds(0, ROWS_T)], acc.at[pl.ds(s * ROWS_T, ROWS_T)]
        )
        plsc.subcore_barrier()

        def start_in(b):
            slot = b % 3
            ha = pltpu.async_copy(
                attr_hbm.at[pl.ds(wid * E_T + b * E_B, E_B)],
                attr_buf.at[slot],
                sem_a,
            )
            hi = pltpu.async_copy(
                col_hbm.at[wid, b], idx_buf.at[slot], sem_i
            )
            return ha, hi

        ones16 = jnp.ones((16,), jnp.float32)
        pending = {0: start_in(0)}
        scat = {}
        for b in range(NB):
            slot = b % 3
            ha, hi = pending.pop(b)
            ha.wait()
            hi.wait()
            if b + 1 < NB:
                pending[b + 1] = start_in(b + 1)

            scat[b] = [
                pltpu.async_copy(
                    attr_buf.at[slot, pl.ds(j * CH, CH)],
                    acc.at[idx_buf.at[slot, j]],
                    ssc,
                    add=True,
                )
                for j in range(NSUB)
            ]

            def count_chunk(j, carry):
                for kk in range(CH // 16):
                    v = idx_buf[slot, j, pl.ds(kk * 16, 16)]
                    plsc.addupdate_scatter(cnt_acc, [v], ones16)
                return carry

            lax.fori_loop(0, NSUB, count_chunk, 0)
            if b - 1 in scat:
                for h in scat.pop(b - 1):
                    h.wait()
        for h in scat.pop(NB - 1):
            h.wait()

        plsc.subcore_barrier()
        pltpu.sync_copy(
            acc.at[pl.ds(s * ROWS_T, ROWS_T)],
            sums_hbm.at[c, pl.ds(s * ROWS_T, ROWS_T)],
        )
        for p in range(N // 1000):
            pltpu.sync_copy(
                cnt_acc.at[pl.ds(p * 1000, 1000)], cnt_hbm.at[p, wid]
            )

    return k(attr2, col4)


def _mlp_pre(x, W1a, b1):
    BR = 1000

    def body(x_ref, w1a, b1r, o_ref):
        o_ref[...] = (
            jnp.dot(x_ref[...], w1a[...], preferred_element_type=jnp.float32)
            + b1r[...]
        )

    return pl.pallas_call(
        body,
        grid=(N // BR,),
        in_specs=[
            pl.BlockSpec((BR, 128), lambda i: (i, 0)),
            pl.BlockSpec((128, 128), lambda i: (0, 0)),
            pl.BlockSpec((1, 128), lambda i: (0, 0)),
        ],
        out_specs=pl.BlockSpec((BR, 128), lambda i: (i, 0)),
        out_shape=jax.ShapeDtypeStruct((N, 128), jnp.float32),
    )(x, W1a, b1.reshape(1, 128))


def _mlp_post(pre, sums2, cnts_t, W1b, W2, b2):
    BR = 1000

    def body(p_ref, s_ref, c_ref, w1b, w2, b2r, o_ref):
        cnt = jnp.sum(c_ref[0], axis=0)
        sm = s_ref[0] + s_ref[1]
        mean = sm / jnp.maximum(cnt, 1.0)[:, None]
        h = jnp.maximum(
            p_ref[...]
            + jnp.dot(mean, w1b[...], preferred_element_type=jnp.float32),
            0.0,
        )
        o_ref[...] = jnp.dot(h, w2[...], preferred_element_type=jnp.float32) + b2r[...]

    return pl.pallas_call(
        body,
        grid=(N // BR,),
        in_specs=[
            pl.BlockSpec((BR, 128), lambda i: (i, 0)),
            pl.BlockSpec((2, BR, DE), lambda i: (0, i, 0)),
            pl.BlockSpec((1, NW, BR), lambda i: (i, 0, 0)),
            pl.BlockSpec((DE, 128), lambda i: (0, 0)),
            pl.BlockSpec((128, 128), lambda i: (0, 0)),
            pl.BlockSpec((1, 128), lambda i: (0, 0)),
        ],
        out_specs=pl.BlockSpec((BR, 128), lambda i: (i, 0)),
        out_shape=jax.ShapeDtypeStruct((N, 128), jnp.float32),
    )(pre, sums2, cnts_t, W1b, W2, b2.reshape(1, 128))


def kernel(x, edge_index, edge_attr, W1, b1, W2, b2):
    col = edge_index[1]
    col4 = col.reshape(NW, NB, NSUB, CH)
    pre = _mlp_pre(x, W1[:128], b1)
    sums2, cnts = _sc_scatter(edge_attr, col4)
    return _mlp_post(pre, sums2, cnts, W1[128:], W2, b2)
